# baseline (speedup 1.0000x reference)
import functools

import jax
import jax.numpy as jnp
from jax import lax
from jax.experimental import pallas as pl
from jax.experimental.pallas import tpu as pltpu
from jax.experimental.pallas import tpu_sc as plsc

_SEQ = 200
_D = 128
_ROWS_PER_WORKER = 8
_NUM_GATHER_WORKERS = _SEQ // _ROWS_PER_WORKER


def _sc_gather_body(x0_hbm, x1_hbm, emb1_hbm, emb2_hbm, out_hbm,
                    idx_v, rows_v, acc_v, idx2_v, row2_v, sem):
    nc = 2
    wid = lax.axis_index("s") * nc + lax.axis_index("c")

    @pl.when(wid < _NUM_GATHER_WORKERS)
    def _gather_emb1():
        base = wid * _ROWS_PER_WORKER
        pltpu.sync_copy(x0_hbm.at[pl.ds(base, _ROWS_PER_WORKER)], idx_v)
        pltpu.async_copy(emb1_hbm.at[idx_v], rows_v, sem).wait()
        for c in range(_D // 16):
            acc = rows_v[0, pl.ds(c * 16, 16)]
            for r in range(1, _ROWS_PER_WORKER):
                acc = acc + rows_v[r, pl.ds(c * 16, 16)]
            acc_v[pl.ds(c * 16, 16)] = acc
        pltpu.sync_copy(acc_v, out_hbm.at[wid])

    @pl.when(wid == _NUM_GATHER_WORKERS)
    def _gather_emb2():
        pltpu.sync_copy(x1_hbm, idx2_v)
        pltpu.async_copy(emb2_hbm.at[idx2_v], row2_v, sem).wait()
        pltpu.sync_copy(row2_v, out_hbm.at[pl.ds(_NUM_GATHER_WORKERS, 1)])


_sc_gather = functools.partial(
    pl.kernel,
    _sc_gather_body,
    out_type=jax.ShapeDtypeStruct((_NUM_GATHER_WORKERS + 1, _D), jnp.float32),
    scratch_types=[
        pltpu.VMEM((_ROWS_PER_WORKER,), jnp.int32),
        pltpu.VMEM((_ROWS_PER_WORKER, _D), jnp.float32),
        pltpu.VMEM((_D,), jnp.float32),
        pltpu.VMEM((1,), jnp.int32),
        pltpu.VMEM((1, _D), jnp.float32),
        pltpu.SemaphoreType.DMA,
    ],
    mesh=plsc.VectorSubcoreMesh(core_axis_name="c", subcore_axis_name="s"),
)()


def _tc_dense_body(stage_ref, w1_ref, b1_ref, out_ref):
    parts = stage_ref[0:_NUM_GATHER_WORKERS, :]
    m = jnp.sum(parts, axis=0, keepdims=True) * (1.0 / _SEQ)
    y1 = lax.dot_general(m, w1_ref[...], (((1,), (1,)), ((), ())),
                         preferred_element_type=jnp.float32)
    y1 = y1 + b1_ref[...]
    v = stage_ref[_NUM_GATHER_WORKERS:_NUM_GATHER_WORKERS + 1, :]
    s = jnp.sum(y1 * v)
    out_ref[...] = jax.nn.sigmoid(s) * jnp.ones((1, _D), jnp.float32)


def kernel(x0, x1, emb1, W1, b1, emb2):
    stage = _sc_gather(x0, x1, emb1, emb2)
    out = pl.pallas_call(
        _tc_dense_body,
        out_shape=jax.ShapeDtypeStruct((1, _D), jnp.float32),
    )(stage, W1, b1.reshape(1, _D))
    return out[0, 0]
